# baseline (speedup 1.0000x reference)
import functools

import jax
import jax.numpy as jnp
from jax import lax
from jax.experimental import pallas as pl
from jax.experimental.pallas import tpu as pltpu
from jax.experimental.pallas import tpu_sc as plsc

B = 4096
L = 200
EMB = 64
NC = 2
NS = 16
NW = NC * NS
BPW = B // NW
IPW = BPW * L
C0 = 104
C1 = L - C0


def _sc_pool_body(x_hbm, table_hbm, out_hbm, xv, rows, acc, sem):
  cid = lax.axis_index("c")
  sid = lax.axis_index("s")
  wid = sid * NC + cid

  pltpu.sync_copy(x_hbm.at[pl.ds(wid * IPW, IPW)], xv)

  def row_body(r, _):
    off = r * L
    g0 = pltpu.async_copy(
        table_hbm.at[xv.at[pl.ds(off, C0)]], rows.at[pl.ds(0, C0)], sem)
    g1 = pltpu.async_copy(
        table_hbm.at[xv.at[pl.ds(off + C0, C1)]], rows.at[pl.ds(C0, C1)], sem)
    g0.wait()
    g1.wait()

    def red(i, carry):
      a0, a1, a2, a3 = carry
      return (a0 + rows[i, pl.ds(0, 16)],
              a1 + rows[i, pl.ds(16, 16)],
              a2 + rows[i, pl.ds(32, 16)],
              a3 + rows[i, pl.ds(48, 16)])

    z = jnp.zeros((16,), jnp.float32)
    a0, a1, a2, a3 = lax.fori_loop(0, L, red, (z, z, z, z))
    acc[r, pl.ds(0, 16)] = a0
    acc[r, pl.ds(16, 16)] = a1
    acc[r, pl.ds(32, 16)] = a2
    acc[r, pl.ds(48, 16)] = a3
    return 0

  lax.fori_loop(0, BPW, row_body, 0)
  pltpu.sync_copy(acc, out_hbm.at[pl.ds(wid * BPW, BPW)])


_sc_pool = pl.kernel(
    _sc_pool_body,
    out_type=jax.ShapeDtypeStruct((B, EMB), jnp.float32),
    mesh=plsc.VectorSubcoreMesh(
        core_axis_name="c", subcore_axis_name="s", num_cores=NC,
        num_subcores=NS),
    scratch_types=[
        pltpu.VMEM((IPW,), jnp.int32),
        pltpu.VMEM((L, EMB), jnp.float32),
        pltpu.VMEM((BPW, EMB), jnp.float32),
        pltpu.SemaphoreType.DMA,
    ],
    compiler_params=pltpu.CompilerParams(use_tc_tiling_on_sc=False),
)


def _tc_linear_body(s_ref, w_ref, b_ref, o_ref):
  s = s_ref[...]
  o = lax.dot_general(s, w_ref[...], (((1,), (1,)), ((), ())),
                      preferred_element_type=jnp.float32)
  o_ref[...] = jnp.maximum(o * (1.0 / L) + b_ref[...], 0.0)


_TC_BLK = 512
_tc_linear = pl.pallas_call(
    _tc_linear_body,
    grid=(B // _TC_BLK,),
    in_specs=[
        pl.BlockSpec((_TC_BLK, EMB), lambda i: (i, 0)),
        pl.BlockSpec((EMB, EMB), lambda i: (0, 0)),
        pl.BlockSpec((1, EMB), lambda i: (0, 0)),
    ],
    out_specs=pl.BlockSpec((_TC_BLK, EMB), lambda i: (i, 0)),
    out_shape=jax.ShapeDtypeStruct((B, EMB), jnp.float32),
)


@jax.jit
def kernel(x, table, W, b):
  x_flat = x.reshape(-1).astype(jnp.int32)
  sums = _sc_pool(x_flat, table)
  return _tc_linear(sums, W, b.reshape(1, EMB))


# speedup vs baseline: 1.1677x; 1.1677x over previous
import functools

import jax
import jax.numpy as jnp
from jax import lax
from jax.experimental import pallas as pl
from jax.experimental.pallas import tpu as pltpu
from jax.experimental.pallas import tpu_sc as plsc

B = 4096
L = 200
EMB = 64
NC = 2
NS = 16
NW = NC * NS
BPW = B // NW
IPW = BPW * L
C0 = 104
C1 = L - C0


_UNROLL = 8


def _sc_pool_body(x_hbm, table_hbm, out_hbm, xv, rows0, rows1, acc, sem0,
                  sem1):
  cid = lax.axis_index("c")
  sid = lax.axis_index("s")
  wid = sid * NC + cid

  pltpu.sync_copy(x_hbm.at[pl.ds(wid * IPW, IPW)], xv)

  def fire(r, buf, sem):
    off = r * L
    pltpu.async_copy(
        table_hbm.at[xv.at[pl.ds(off, C0)]], buf.at[pl.ds(0, C0)], sem)
    pltpu.async_copy(
        table_hbm.at[xv.at[pl.ds(off + C0, C1)]], buf.at[pl.ds(C0, C1)], sem)

  def drain(buf, sem):
    pltpu.make_async_copy(
        table_hbm.at[xv.at[pl.ds(0, C0)]], buf.at[pl.ds(0, C0)], sem).wait()
    pltpu.make_async_copy(
        table_hbm.at[xv.at[pl.ds(0, C1)]], buf.at[pl.ds(C0, C1)], sem).wait()

  def reduce_into(r, buf):
    def red(i, carry):
      a0, a1, a2, a3 = carry
      base = i * _UNROLL
      for u in range(_UNROLL):
        a0 = a0 + buf[base + u, pl.ds(0, 16)]
        a1 = a1 + buf[base + u, pl.ds(16, 16)]
        a2 = a2 + buf[base + u, pl.ds(32, 16)]
        a3 = a3 + buf[base + u, pl.ds(48, 16)]
      return (a0, a1, a2, a3)

    z = jnp.zeros((16,), jnp.float32)
    a0, a1, a2, a3 = lax.fori_loop(0, L // _UNROLL, red, (z, z, z, z))
    acc[r, pl.ds(0, 16)] = a0
    acc[r, pl.ds(16, 16)] = a1
    acc[r, pl.ds(32, 16)] = a2
    acc[r, pl.ds(48, 16)] = a3

  fire(0, rows0, sem0)
  fire(1, rows1, sem1)

  def pipe_body(k, _):
    r = 2 * k
    drain(rows0, sem0)
    reduce_into(r, rows0)
    fire(r + 2, rows0, sem0)
    drain(rows1, sem1)
    reduce_into(r + 1, rows1)
    fire(r + 3, rows1, sem1)
    return 0

  lax.fori_loop(0, BPW // 2 - 1, pipe_body, 0)
  drain(rows0, sem0)
  reduce_into(BPW - 2, rows0)
  drain(rows1, sem1)
  reduce_into(BPW - 1, rows1)

  pltpu.sync_copy(acc, out_hbm.at[pl.ds(wid * BPW, BPW)])


_sc_pool = pl.kernel(
    _sc_pool_body,
    out_type=jax.ShapeDtypeStruct((B, EMB), jnp.float32),
    mesh=plsc.VectorSubcoreMesh(
        core_axis_name="c", subcore_axis_name="s", num_cores=NC,
        num_subcores=NS),
    scratch_types=[
        pltpu.VMEM((IPW,), jnp.int32),
        pltpu.VMEM((L, EMB), jnp.float32),
        pltpu.VMEM((L, EMB), jnp.float32),
        pltpu.VMEM((BPW, EMB), jnp.float32),
        pltpu.SemaphoreType.DMA,
        pltpu.SemaphoreType.DMA,
    ],
    compiler_params=pltpu.CompilerParams(use_tc_tiling_on_sc=False),
)


def _tc_linear_body(s_ref, w_ref, b_ref, o_ref):
  s = s_ref[...]
  o = lax.dot_general(s, w_ref[...], (((1,), (1,)), ((), ())),
                      preferred_element_type=jnp.float32)
  o_ref[...] = jnp.maximum(o * (1.0 / L) + b_ref[...], 0.0)


_TC_BLK = 512
_tc_linear = pl.pallas_call(
    _tc_linear_body,
    grid=(B // _TC_BLK,),
    in_specs=[
        pl.BlockSpec((_TC_BLK, EMB), lambda i: (i, 0)),
        pl.BlockSpec((EMB, EMB), lambda i: (0, 0)),
        pl.BlockSpec((1, EMB), lambda i: (0, 0)),
    ],
    out_specs=pl.BlockSpec((_TC_BLK, EMB), lambda i: (i, 0)),
    out_shape=jax.ShapeDtypeStruct((B, EMB), jnp.float32),
)


@jax.jit
def kernel(x, table, W, b):
  x_flat = x.reshape(-1).astype(jnp.int32)
  sums = _sc_pool(x_flat, table)
  return _tc_linear(sums, W, b.reshape(1, EMB))


# speedup vs baseline: 1.2637x; 1.0821x over previous
import functools

import jax
import jax.numpy as jnp
from jax import lax
from jax.experimental import pallas as pl
from jax.experimental.pallas import tpu as pltpu
from jax.experimental.pallas import tpu_sc as plsc

B = 4096
L = 200
EMB = 64
NC = 2
NS = 16
NW = NC * NS
BPW = B // NW
IPW = BPW * L
C0 = 104
C1 = L - C0


_UNROLL = 8


def _sc_pool_body(x_hbm, table_hbm, out_hbm, xv, rows0, rows1, acc, sem0,
                  sem1):
  cid = lax.axis_index("c")
  sid = lax.axis_index("s")
  wid = sid * NC + cid

  pltpu.sync_copy(x_hbm.at[pl.ds(wid * IPW, IPW)], xv)

  def dbl(i, _):
    xv[pl.ds(i * 16, 16)] = xv[pl.ds(i * 16, 16)] * 2
    return 0

  lax.fori_loop(0, IPW // 16, dbl, 0)

  def fire(r, buf, sem):
    off = r * L
    pltpu.async_copy(
        table_hbm.at[xv.at[pl.ds(off, C0)]], buf.at[pl.ds(0, C0)], sem)
    pltpu.async_copy(
        table_hbm.at[xv.at[pl.ds(off + C0, C1)]], buf.at[pl.ds(C0, C1)], sem)

  def drain(buf, sem):
    pltpu.make_async_copy(
        table_hbm.at[xv.at[pl.ds(0, C0)]], buf.at[pl.ds(0, C0)], sem).wait()
    pltpu.make_async_copy(
        table_hbm.at[xv.at[pl.ds(0, C1)]], buf.at[pl.ds(C0, C1)], sem).wait()

  def reduce_into(r, buf):
    def red(i, carry):
      a0, a1, a2, a3 = carry
      base = i * _UNROLL
      for u in range(_UNROLL):
        a0 = a0 + buf[base + u, pl.ds(0, 16)]
        a1 = a1 + buf[base + u, pl.ds(16, 16)]
        a2 = a2 + buf[base + u, pl.ds(32, 16)]
        a3 = a3 + buf[base + u, pl.ds(48, 16)]
      return (a0, a1, a2, a3)

    z = jnp.zeros((16,), jnp.float32)
    a0, a1, a2, a3 = lax.fori_loop(0, L // _UNROLL, red, (z, z, z, z))
    acc[r, pl.ds(0, 16)] = a0
    acc[r, pl.ds(16, 16)] = a1
    acc[r, pl.ds(32, 16)] = a2
    acc[r, pl.ds(48, 16)] = a3

  fire(0, rows0, sem0)
  fire(1, rows1, sem1)

  def pipe_body(k, _):
    r = 2 * k
    drain(rows0, sem0)
    reduce_into(r, rows0)
    fire(r + 2, rows0, sem0)
    drain(rows1, sem1)
    reduce_into(r + 1, rows1)
    fire(r + 3, rows1, sem1)
    return 0

  lax.fori_loop(0, BPW // 2 - 1, pipe_body, 0)
  drain(rows0, sem0)
  reduce_into(BPW - 2, rows0)
  drain(rows1, sem1)
  reduce_into(BPW - 1, rows1)

  pltpu.sync_copy(acc, out_hbm.at[pl.ds(wid * BPW, BPW)])


_sc_pool = pl.kernel(
    _sc_pool_body,
    out_type=jax.ShapeDtypeStruct((B, EMB), jnp.float32),
    mesh=plsc.VectorSubcoreMesh(
        core_axis_name="c", subcore_axis_name="s", num_cores=NC,
        num_subcores=NS),
    scratch_types=[
        pltpu.VMEM((IPW,), jnp.int32),
        pltpu.VMEM((L, EMB), jnp.float32),
        pltpu.VMEM((L, EMB), jnp.float32),
        pltpu.VMEM((BPW, EMB), jnp.float32),
        pltpu.SemaphoreType.DMA,
        pltpu.SemaphoreType.DMA,
    ],
    compiler_params=pltpu.CompilerParams(use_tc_tiling_on_sc=False),
)


def _tc_linear_body(s_ref, w_ref, b_ref, o_ref):
  s = s_ref[...]
  o = lax.dot_general(s, w_ref[...], (((1,), (1,)), ((), ())),
                      preferred_element_type=jnp.float32)
  o_ref[...] = jnp.maximum(o * (1.0 / L) + b_ref[...], 0.0)


_TC_BLK = 512
_tc_linear = pl.pallas_call(
    _tc_linear_body,
    grid=(B // _TC_BLK,),
    in_specs=[
        pl.BlockSpec((_TC_BLK, EMB), lambda i: (i, 0)),
        pl.BlockSpec((EMB, EMB), lambda i: (0, 0)),
        pl.BlockSpec((1, EMB), lambda i: (0, 0)),
    ],
    out_specs=pl.BlockSpec((_TC_BLK, EMB), lambda i: (i, 0)),
    out_shape=jax.ShapeDtypeStruct((B, EMB), jnp.float32),
)


@jax.jit
def kernel(x, table, W, b):
  x_flat = x.reshape(-1).astype(jnp.int32)
  tp = jnp.pad(table, ((0, 0), (0, EMB)))
  tp2 = tp.reshape(2 * tp.shape[0], EMB)
  sums = _sc_pool(x_flat, tp2)
  return _tc_linear(sums, W, b.reshape(1, EMB))


# speedup vs baseline: 1.7410x; 1.3778x over previous
import functools

import jax
import jax.numpy as jnp
from jax import lax
from jax.experimental import pallas as pl
from jax.experimental.pallas import tpu as pltpu
from jax.experimental.pallas import tpu_sc as plsc

B = 4096
L = 200
EMB = 64
NC = 2
NS = 16
NW = NC * NS
BPW = B // NW
IPW = BPW * L
C0 = 104
C1 = L - C0


_UNROLL = 8


def _sc_pool_body(x_hbm, table_hbm, out_hbm, xv, rows0, rows1, acc, sem0,
                  sem1):
  cid = lax.axis_index("c")
  sid = lax.axis_index("s")
  wid = sid * NC + cid

  pltpu.sync_copy(x_hbm.at[pl.ds(wid * IPW, IPW)], xv)

  def dbl(i, _):
    xv[pl.ds(i * 16, 16)] = xv[pl.ds(i * 16, 16)] * 2
    return 0

  lax.fori_loop(0, IPW // 16, dbl, 0)

  def fire(r, buf, sem):
    off = r * L
    pltpu.async_copy(
        table_hbm.at[xv.at[pl.ds(off, C0)]], buf.at[pl.ds(0, C0)], sem)
    pltpu.async_copy(
        table_hbm.at[xv.at[pl.ds(off + C0, C1)]], buf.at[pl.ds(C0, C1)], sem)

  def drain(buf, sem):
    pltpu.make_async_copy(
        table_hbm.at[xv.at[pl.ds(0, C0)]], buf.at[pl.ds(0, C0)], sem).wait()
    pltpu.make_async_copy(
        table_hbm.at[xv.at[pl.ds(0, C1)]], buf.at[pl.ds(C0, C1)], sem).wait()

  def reduce_into(r, buf):
    def red(i, carry):
      a0, a1, a2, a3 = carry
      base = i * _UNROLL
      for u in range(_UNROLL):
        a0 = a0 + buf[base + u, pl.ds(0, 16)]
        a1 = a1 + buf[base + u, pl.ds(16, 16)]
        a2 = a2 + buf[base + u, pl.ds(32, 16)]
        a3 = a3 + buf[base + u, pl.ds(48, 16)]
      return (a0, a1, a2, a3)

    z = jnp.zeros((16,), jnp.float32)
    a0, a1, a2, a3 = lax.fori_loop(0, L // _UNROLL, red, (z, z, z, z))
    acc[r, pl.ds(0, 16)] = a0
    acc[r, pl.ds(16, 16)] = a1
    acc[r, pl.ds(32, 16)] = a2
    acc[r, pl.ds(48, 16)] = a3

  fire(0, rows0, sem0)
  fire(1, rows1, sem1)

  def pipe_body(k, _):
    r = 2 * k
    drain(rows0, sem0)
    reduce_into(r, rows0)
    fire(r + 2, rows0, sem0)
    drain(rows1, sem1)
    reduce_into(r + 1, rows1)
    fire(r + 3, rows1, sem1)
    return 0

  lax.fori_loop(0, BPW // 2 - 1, pipe_body, 0)
  drain(rows0, sem0)
  reduce_into(BPW - 2, rows0)
  drain(rows1, sem1)
  reduce_into(BPW - 1, rows1)

  pltpu.sync_copy(acc, out_hbm.at[pl.ds(wid * BPW, BPW)])


_sc_pool = pl.kernel(
    _sc_pool_body,
    out_type=jax.ShapeDtypeStruct((B, EMB), jnp.float32),
    mesh=plsc.VectorSubcoreMesh(
        core_axis_name="c", subcore_axis_name="s", num_cores=NC,
        num_subcores=NS),
    scratch_types=[
        pltpu.VMEM((IPW,), jnp.int32),
        pltpu.VMEM((L, EMB), jnp.float32),
        pltpu.VMEM((L, EMB), jnp.float32),
        pltpu.VMEM((BPW, EMB), jnp.float32),
        pltpu.SemaphoreType.DMA,
        pltpu.SemaphoreType.DMA,
    ],
    compiler_params=pltpu.CompilerParams(use_tc_tiling_on_sc=False),
)


_DT_BC = 4096


def _tc_detile_body(t_ref, o_ref):
  t = jnp.transpose(t_ref[...], (1, 0))
  o_ref[...] = jnp.concatenate(
      [t, jnp.zeros((_DT_BC, EMB), jnp.float32)], axis=1)


def _tc_detile(tab_t):
  vocab = tab_t.shape[1]
  grid = (vocab + _DT_BC - 1) // _DT_BC
  return pl.pallas_call(
      _tc_detile_body,
      grid=(grid,),
      in_specs=[pl.BlockSpec((EMB, _DT_BC), lambda i: (0, i))],
      out_specs=pl.BlockSpec((_DT_BC, 2 * EMB), lambda i: (i, 0)),
      out_shape=jax.ShapeDtypeStruct((vocab, 2 * EMB), jnp.float32),
  )(tab_t)


def _tc_linear_body(s_ref, w_ref, b_ref, o_ref):
  s = s_ref[...]
  o = lax.dot_general(s, w_ref[...], (((1,), (1,)), ((), ())),
                      preferred_element_type=jnp.float32)
  o_ref[...] = jnp.maximum(o * (1.0 / L) + b_ref[...], 0.0)


_TC_BLK = 512
_tc_linear = pl.pallas_call(
    _tc_linear_body,
    grid=(B // _TC_BLK,),
    in_specs=[
        pl.BlockSpec((_TC_BLK, EMB), lambda i: (i, 0)),
        pl.BlockSpec((EMB, EMB), lambda i: (0, 0)),
        pl.BlockSpec((1, EMB), lambda i: (0, 0)),
    ],
    out_specs=pl.BlockSpec((_TC_BLK, EMB), lambda i: (i, 0)),
    out_shape=jax.ShapeDtypeStruct((B, EMB), jnp.float32),
)


@jax.jit
def kernel(x, table, W, b):
  x_flat = x.reshape(-1).astype(jnp.int32)
  tp = _tc_detile(table.T)
  tp2 = tp.reshape(2 * tp.shape[0], EMB)
  sums = _sc_pool(x_flat, tp2)
  return _tc_linear(sums, W, b.reshape(1, EMB))
